# baseline (speedup 1.0000x reference)
import functools

import jax
import jax.numpy as jnp
from jax import lax
from jax.experimental import pallas as pl
from jax.experimental.pallas import tpu as pltpu
from jax.experimental.pallas import tpu_sc as plsc

D_FEAT = 128
D_EDGE = 16
OUT_DIM = 128
_SC_BLOCK = 128


def _nodeproj_body(nf, wsl, wsh, wdl, wdh, psl, psh, pdl, pdh):
    x = nf[...]
    psl[...] = jnp.dot(x, wsl[...], preferred_element_type=jnp.float32)
    psh[...] = jnp.dot(x, wsh[...], preferred_element_type=jnp.float32)
    pdl[...] = jnp.dot(x, wdl[...], preferred_element_type=jnp.float32)
    pdh[...] = jnp.dot(x, wdh[...], preferred_element_type=jnp.float32)


def _node_projections(node_feat, W_s, W_d):
    N = node_feat.shape[0]
    BLK = 2000
    H = OUT_DIM // 2
    halves = [W_s[:, :H], W_s[:, H:], W_d[:, :H], W_d[:, H:]]
    return pl.pallas_call(
        _nodeproj_body,
        grid=(N // BLK,),
        in_specs=[pl.BlockSpec((BLK, D_FEAT), lambda i: (i, 0))]
        + [pl.BlockSpec((D_FEAT, H), lambda i: (0, 0))] * 4,
        out_specs=[pl.BlockSpec((BLK, H), lambda i: (i, 0))] * 4,
        out_shape=[jax.ShapeDtypeStruct((N, H), jnp.float32)] * 4,
    )(node_feat, *halves)


def _make_gather_sum(E, N):
    info = plsc.get_sparse_core_info()
    NC, NS = info.num_cores, info.num_subcores
    B = _SC_BLOCK
    nblk = E // B
    HALF = OUT_DIM // 2
    RPW = N // NS
    mesh = plsc.VectorSubcoreMesh(core_axis_name="c", subcore_axis_name="s")

    @functools.partial(
        pl.kernel,
        mesh=mesh,
        compiler_params=pltpu.CompilerParams(use_tc_tiling_on_sc=False),
        out_type=jax.ShapeDtypeStruct((E, OUT_DIM), jnp.float32),
        scratch_types=[
            pltpu.VMEM_SHARED((N, HALF), jnp.float32),
            pltpu.VMEM_SHARED((N, HALF), jnp.float32),
            pltpu.VMEM((B,), jnp.int32),
            pltpu.VMEM((B,), jnp.int32),
            pltpu.VMEM((B, HALF), jnp.float32),
            pltpu.VMEM((B, HALF), jnp.float32),
            pltpu.SemaphoreType.DMA,
            pltpu.SemaphoreType.DMA,
        ],
    )
    def gather_sum(psl_hbm, psh_hbm, pdl_hbm, pdh_hbm, src_hbm, dst_hbm,
                   g_hbm, ps_sh, pd_sh, idx_s, idx_d, buf_s, buf_d,
                   sem_s, sem_d):
        c = lax.axis_index("c")
        sub = lax.axis_index("s")
        col0 = c * HALF
        rows = pl.ds(sub * RPW, RPW)

        @pl.when(c == 0)
        def _():
            pltpu.sync_copy(psl_hbm.at[rows], ps_sh.at[rows])
            pltpu.sync_copy(pdl_hbm.at[rows], pd_sh.at[rows])

        @pl.when(c == 1)
        def _():
            pltpu.sync_copy(psh_hbm.at[rows], ps_sh.at[rows])
            pltpu.sync_copy(pdh_hbm.at[rows], pd_sh.at[rows])

        plsc.subcore_barrier()

        my_n = (nblk - sub + NS - 1) // NS

        def blk_body(i, carry):
            base = (sub + i * NS) * B
            pltpu.sync_copy(src_hbm.at[pl.ds(base, B)], idx_s)
            pltpu.sync_copy(dst_hbm.at[pl.ds(base, B)], idx_d)
            c1 = pltpu.async_copy(ps_sh.at[idx_s], buf_s, sem_s)
            c2 = pltpu.async_copy(pd_sh.at[idx_d], buf_d, sem_d)
            c1.wait()
            c2.wait()

            def row_body(r, rcarry):
                for cc in range(HALF // 16):
                    sl = pl.ds(cc * 16, 16)
                    plsc.addupdate(buf_s.at[r, sl], buf_d[r, sl])
                return rcarry

            lax.fori_loop(0, B, row_body, 0)
            pltpu.sync_copy(buf_s, g_hbm.at[pl.ds(base, B), pl.ds(col0, HALF)])
            return carry

        lax.fori_loop(0, my_n, blk_body, 0)

    return gather_sum


def _edge_body(ea, we, bb, g, out):
    out[...] = (g[...]
                + jnp.dot(ea[...], we[...], preferred_element_type=jnp.float32)
                + bb[...])


def _edge_combine(edge_attr, W_e, b2d, G):
    E = edge_attr.shape[0]
    BLK = 4000
    return pl.pallas_call(
        _edge_body,
        grid=(E // BLK,),
        in_specs=[
            pl.BlockSpec((BLK, D_EDGE), lambda i: (i, 0)),
            pl.BlockSpec((D_EDGE, OUT_DIM), lambda i: (0, 0)),
            pl.BlockSpec((1, OUT_DIM), lambda i: (0, 0)),
            pl.BlockSpec((BLK, OUT_DIM), lambda i: (i, 0)),
        ],
        out_specs=pl.BlockSpec((BLK, OUT_DIM), lambda i: (i, 0)),
        out_shape=jax.ShapeDtypeStruct((E, OUT_DIM), jnp.float32),
    )(edge_attr, W_e, b2d, G)


def kernel(node_feat, edge_attr, edge_index, W, b):
    W_e = W[:D_EDGE]
    W_s = W[D_EDGE:D_EDGE + D_FEAT]
    W_d = W[D_EDGE + D_FEAT:]
    src = edge_index[0]
    dst = edge_index[1]
    psl, psh, pdl, pdh = _node_projections(node_feat, W_s, W_d)
    G = _make_gather_sum(edge_attr.shape[0], node_feat.shape[0])(
        psl, psh, pdl, pdh, src, dst)
    return _edge_combine(edge_attr, W_e, b.reshape(1, OUT_DIM), G)


# speedup vs baseline: 1.4646x; 1.4646x over previous
import functools

import jax
import jax.numpy as jnp
from jax import lax
from jax.experimental import pallas as pl
from jax.experimental.pallas import tpu as pltpu
from jax.experimental.pallas import tpu_sc as plsc

D_FEAT = 128
D_EDGE = 16
OUT_DIM = 128
_SC_BLOCK = 128


def _nodeproj_body(nf, ws, wd, ps, pd):
    x = nf[...]
    ps[...] = jnp.dot(x, ws[...], preferred_element_type=jnp.float32)
    pd[...] = jnp.dot(x, wd[...], preferred_element_type=jnp.float32)


def _node_projections(node_feat, W_s, W_d):
    N = node_feat.shape[0]
    BLK = 2000
    return pl.pallas_call(
        _nodeproj_body,
        grid=(N // BLK,),
        in_specs=[
            pl.BlockSpec((BLK, D_FEAT), lambda i: (i, 0)),
            pl.BlockSpec((D_FEAT, OUT_DIM), lambda i: (0, 0)),
            pl.BlockSpec((D_FEAT, OUT_DIM), lambda i: (0, 0)),
        ],
        out_specs=[
            pl.BlockSpec((BLK, OUT_DIM), lambda i: (i, 0)),
            pl.BlockSpec((BLK, OUT_DIM), lambda i: (i, 0)),
        ],
        out_shape=[
            jax.ShapeDtypeStruct((N, OUT_DIM), jnp.float32),
            jax.ShapeDtypeStruct((N, OUT_DIM), jnp.float32),
        ],
    )(node_feat, W_s, W_d)


def _make_gather_sum(E):
    info = plsc.get_sparse_core_info()
    NC, NS = info.num_cores, info.num_subcores
    NW = NC * NS
    B = _SC_BLOCK
    nblk = E // B
    mesh = plsc.VectorSubcoreMesh(core_axis_name="c", subcore_axis_name="s")

    @functools.partial(
        pl.kernel,
        mesh=mesh,
        out_type=jax.ShapeDtypeStruct((E, OUT_DIM), jnp.float32),
        scratch_types=[
            pltpu.VMEM((2, B), jnp.int32),
            pltpu.VMEM((2, B), jnp.int32),
            pltpu.VMEM((B, OUT_DIM), jnp.float32),
            pltpu.VMEM((B, OUT_DIM), jnp.float32),
            pltpu.VMEM((B, OUT_DIM), jnp.float32),
            pltpu.VMEM((B, OUT_DIM), jnp.float32),
            pltpu.VMEM((B, OUT_DIM), jnp.float32),
            pltpu.VMEM((B, OUT_DIM), jnp.float32),
            pltpu.SemaphoreType.DMA,
            pltpu.SemaphoreType.DMA,
            pltpu.SemaphoreType.DMA,
            pltpu.SemaphoreType.DMA,
        ],
    )
    def gather_sum(ps_hbm, pd_hbm, src_hbm, dst_hbm, g_hbm,
                   idx_s, idx_d, buf_s0, buf_s1, buf_d0, buf_d1,
                   buf_o0, buf_o1, sem_g0, sem_g1, sem_w0, sem_w1):
        wid = lax.axis_index("s") * NC + lax.axis_index("c")
        my_n = (nblk - wid + NW - 1) // NW
        bufs = ((buf_s0, buf_d0, buf_o0, sem_g0, sem_w0),
                (buf_s1, buf_d1, buf_o1, sem_g1, sem_w1))

        def issue(slot, i, guard):
            bs, bd, _, sg, _ = bufs[slot]

            def _go():
                base = (wid + i * NW) * B
                pltpu.sync_copy(src_hbm.at[pl.ds(base, B)], idx_s.at[slot])
                pltpu.sync_copy(dst_hbm.at[pl.ds(base, B)], idx_d.at[slot])
                pltpu.async_copy(ps_hbm.at[idx_s.at[slot]], bs, sg)
                pltpu.async_copy(pd_hbm.at[idx_d.at[slot]], bd, sg)

            if guard:
                pl.when(i < my_n)(_go)
            else:
                _go()

        def finish(slot, i, wait_prev_wb):
            bs, bd, bo, sg, sw = bufs[slot]

            @pl.when(i < my_n)
            def _():
                base = (wid + i * NW) * B
                pltpu.make_async_copy(ps_hbm.at[pl.ds(0, B)], bs, sg).wait()
                pltpu.make_async_copy(pd_hbm.at[pl.ds(0, B)], bd, sg).wait()
                if wait_prev_wb:
                    pltpu.make_async_copy(bo, g_hbm.at[pl.ds(0, B)],
                                          sw).wait()

                def row_body(r, rcarry):
                    for c in range(OUT_DIM // 16):
                        sl = pl.ds(c * 16, 16)
                        bo[r, sl] = bs[r, sl] + bd[r, sl]
                    return rcarry

                lax.fori_loop(0, B, row_body, 0)
                pltpu.async_copy(bo, g_hbm.at[pl.ds(base, B)], sw)

        issue(0, 0, guard=False)
        issue(1, 1, guard=False)
        finish(0, 0, wait_prev_wb=False)
        issue(0, 2, guard=True)
        finish(1, 1, wait_prev_wb=False)
        issue(1, 3, guard=True)

        def pair_body(p, carry):
            i0 = p * 2
            finish(0, i0, wait_prev_wb=True)
            issue(0, i0 + 2, guard=True)
            finish(1, i0 + 1, wait_prev_wb=True)
            issue(1, i0 + 3, guard=True)
            return carry

        lax.fori_loop(1, (nblk // NW + 3) // 2, pair_body, 0)

    return gather_sum


def _edge_body(ea, we, bb, g, out):
    out[...] = (g[...]
                + jnp.dot(ea[...], we[...], preferred_element_type=jnp.float32)
                + bb[...])


def _edge_combine(edge_attr, W_e, b2d, G):
    E = edge_attr.shape[0]
    BLK = 4000
    return pl.pallas_call(
        _edge_body,
        grid=(E // BLK,),
        in_specs=[
            pl.BlockSpec((BLK, D_EDGE), lambda i: (i, 0)),
            pl.BlockSpec((D_EDGE, OUT_DIM), lambda i: (0, 0)),
            pl.BlockSpec((1, OUT_DIM), lambda i: (0, 0)),
            pl.BlockSpec((BLK, OUT_DIM), lambda i: (i, 0)),
        ],
        out_specs=pl.BlockSpec((BLK, OUT_DIM), lambda i: (i, 0)),
        out_shape=jax.ShapeDtypeStruct((E, OUT_DIM), jnp.float32),
    )(edge_attr, W_e, b2d, G)


def kernel(node_feat, edge_attr, edge_index, W, b):
    W_e = W[:D_EDGE]
    W_s = W[D_EDGE:D_EDGE + D_FEAT]
    W_d = W[D_EDGE + D_FEAT:]
    src = edge_index[0]
    dst = edge_index[1]
    ps, pd = _node_projections(node_feat, W_s, W_d)
    G = _make_gather_sum(edge_attr.shape[0])(ps, pd, src, dst)
    return _edge_combine(edge_attr, W_e, b.reshape(1, OUT_DIM), G)


# speedup vs baseline: 1.5014x; 1.0251x over previous
import functools

import jax
import jax.numpy as jnp
from jax import lax
from jax.experimental import pallas as pl
from jax.experimental.pallas import tpu as pltpu
from jax.experimental.pallas import tpu_sc as plsc

D_FEAT = 128
D_EDGE = 16
OUT_DIM = 128
_SC_BLOCK = 128


def _nodeproj_body(nf, ws, wd, ps, pd):
    x = nf[...]
    ps[...] = jnp.dot(x, ws[...], preferred_element_type=jnp.float32)
    pd[...] = jnp.dot(x, wd[...], preferred_element_type=jnp.float32)


def _node_projections(node_feat, W_s, W_d):
    N = node_feat.shape[0]
    BLK = 2000
    return pl.pallas_call(
        _nodeproj_body,
        grid=(N // BLK,),
        in_specs=[
            pl.BlockSpec((BLK, D_FEAT), lambda i: (i, 0)),
            pl.BlockSpec((D_FEAT, OUT_DIM), lambda i: (0, 0)),
            pl.BlockSpec((D_FEAT, OUT_DIM), lambda i: (0, 0)),
        ],
        out_specs=[
            pl.BlockSpec((BLK, OUT_DIM), lambda i: (i, 0)),
            pl.BlockSpec((BLK, OUT_DIM), lambda i: (i, 0)),
        ],
        out_shape=[
            jax.ShapeDtypeStruct((N, OUT_DIM), jnp.float32),
            jax.ShapeDtypeStruct((N, OUT_DIM), jnp.float32),
        ],
    )(node_feat, W_s, W_d)


def _make_gather_sum(E):
    info = plsc.get_sparse_core_info()
    NC, NS = info.num_cores, info.num_subcores
    NW = NC * NS
    B = _SC_BLOCK
    nblk = E // B
    mesh = plsc.VectorSubcoreMesh(core_axis_name="c", subcore_axis_name="s")

    Q, R = divmod(nblk, 32)
    NMAX = Q + (1 if R else 0)

    @functools.partial(
        pl.kernel,
        mesh=mesh,
        out_type=jax.ShapeDtypeStruct((E, OUT_DIM), jnp.float32),
        scratch_types=[
            pltpu.VMEM((NMAX * B,), jnp.int32),
            pltpu.VMEM((NMAX * B,), jnp.int32),
            pltpu.VMEM((B, OUT_DIM), jnp.float32),
            pltpu.VMEM((B, OUT_DIM), jnp.float32),
            pltpu.VMEM((B, OUT_DIM), jnp.float32),
            pltpu.VMEM((B, OUT_DIM), jnp.float32),
            pltpu.VMEM((B, OUT_DIM), jnp.float32),
            pltpu.VMEM((B, OUT_DIM), jnp.float32),
            pltpu.SemaphoreType.DMA,
            pltpu.SemaphoreType.DMA,
            pltpu.SemaphoreType.DMA,
            pltpu.SemaphoreType.DMA,
        ],
    )
    def gather_sum(ps_hbm, pd_hbm, src_hbm, dst_hbm, g_hbm,
                   idx_s, idx_d, buf_s0, buf_s1, buf_d0, buf_d1,
                   buf_o0, buf_o1, sem_g0, sem_g1, sem_w0, sem_w1):
        wid = lax.axis_index("s") * NC + lax.axis_index("c")
        my_n = Q + jnp.where(wid < R, 1, 0)
        start = wid * Q + jnp.minimum(wid, R)
        estart = start * B
        pltpu.sync_copy(src_hbm.at[pl.ds(estart, Q * B)],
                        idx_s.at[pl.ds(0, Q * B)])
        pltpu.sync_copy(dst_hbm.at[pl.ds(estart, Q * B)],
                        idx_d.at[pl.ds(0, Q * B)])

        @pl.when(my_n > Q)
        def _():
            pltpu.sync_copy(src_hbm.at[pl.ds(estart + Q * B, B)],
                            idx_s.at[pl.ds(Q * B, B)])
            pltpu.sync_copy(dst_hbm.at[pl.ds(estart + Q * B, B)],
                            idx_d.at[pl.ds(Q * B, B)])

        bufs = ((buf_s0, buf_d0, buf_o0, sem_g0, sem_w0),
                (buf_s1, buf_d1, buf_o1, sem_g1, sem_w1))

        def issue(slot, i, guard):
            bs, bd, _, sg, _ = bufs[slot]

            def _go():
                pltpu.async_copy(ps_hbm.at[idx_s.at[pl.ds(i * B, B)]], bs, sg)
                pltpu.async_copy(pd_hbm.at[idx_d.at[pl.ds(i * B, B)]], bd, sg)

            if guard:
                pl.when(i < my_n)(_go)
            else:
                _go()

        def finish(slot, i, wait_prev_wb):
            bs, bd, bo, sg, sw = bufs[slot]

            @pl.when(i < my_n)
            def _():
                base = (start + i) * B
                pltpu.make_async_copy(ps_hbm.at[pl.ds(0, B)], bs, sg).wait()
                pltpu.make_async_copy(pd_hbm.at[pl.ds(0, B)], bd, sg).wait()
                if wait_prev_wb:
                    pltpu.make_async_copy(bo, g_hbm.at[pl.ds(0, B)],
                                          sw).wait()

                def row_body(r, rcarry):
                    for c in range(OUT_DIM // 16):
                        sl = pl.ds(c * 16, 16)
                        bo[r, sl] = bs[r, sl] + bd[r, sl]
                    return rcarry

                lax.fori_loop(0, B, row_body, 0)
                pltpu.async_copy(bo, g_hbm.at[pl.ds(base, B)], sw)

        issue(0, 0, guard=False)
        issue(1, 1, guard=False)
        finish(0, 0, wait_prev_wb=False)
        issue(0, 2, guard=True)
        finish(1, 1, wait_prev_wb=False)
        issue(1, 3, guard=True)

        def pair_body(p, carry):
            i0 = p * 2
            finish(0, i0, wait_prev_wb=True)
            issue(0, i0 + 2, guard=True)
            finish(1, i0 + 1, wait_prev_wb=True)
            issue(1, i0 + 3, guard=True)
            return carry

        lax.fori_loop(1, (NMAX + 1) // 2 + 1, pair_body, 0)

    return gather_sum


def _edge_body(ea, we, bb, g, out):
    out[...] = (g[...]
                + jnp.dot(ea[...], we[...], preferred_element_type=jnp.float32)
                + bb[...])


def _edge_combine(edge_attr, W_e, b2d, G):
    E = edge_attr.shape[0]
    BLK = 4000
    return pl.pallas_call(
        _edge_body,
        grid=(E // BLK,),
        in_specs=[
            pl.BlockSpec((BLK, D_EDGE), lambda i: (i, 0)),
            pl.BlockSpec((D_EDGE, OUT_DIM), lambda i: (0, 0)),
            pl.BlockSpec((1, OUT_DIM), lambda i: (0, 0)),
            pl.BlockSpec((BLK, OUT_DIM), lambda i: (i, 0)),
        ],
        out_specs=pl.BlockSpec((BLK, OUT_DIM), lambda i: (i, 0)),
        out_shape=jax.ShapeDtypeStruct((E, OUT_DIM), jnp.float32),
    )(edge_attr, W_e, b2d, G)


def kernel(node_feat, edge_attr, edge_index, W, b):
    W_e = W[:D_EDGE]
    W_s = W[D_EDGE:D_EDGE + D_FEAT]
    W_d = W[D_EDGE + D_FEAT:]
    src = edge_index[0]
    dst = edge_index[1]
    ps, pd = _node_projections(node_feat, W_s, W_d)
    G = _make_gather_sum(edge_attr.shape[0])(ps, pd, src, dst)
    return _edge_combine(edge_attr, W_e, b.reshape(1, OUT_DIM), G)
